# baseline (speedup 1.0000x reference)
import functools

import jax
import jax.numpy as jnp
from jax.experimental import pallas as pl
from jax.experimental.pallas import tpu as pltpu

_EPS = 1e-5


def _conv_acc(xt, w_ref, shifts, m):
    acc = jnp.dot(xt[shifts[0]:shifts[0] + m, :], w_ref[0],
                  preferred_element_type=jnp.float32)
    for t, s in enumerate(shifts[1:], start=1):
        acc = acc + jnp.dot(xt[s:s + m, :], w_ref[t],
                            preferred_element_type=jnp.float32)
    return acc


def _stats_kernel(x_ref, w_ref, mask_ref, s_ref, *, shifts, m):
    xt = jnp.transpose(x_ref[0], (1, 0))
    acc = _conv_acc(xt, w_ref, shifts, m)
    ym = acc * mask_ref[...]
    s1 = jnp.sum(ym, axis=0, keepdims=True)
    s2 = jnp.sum(ym * acc, axis=0, keepdims=True)
    s_ref[0] = jnp.concatenate([s1, s2], axis=0)


def _conv_bn_relu_kernel(x_ref, w_ref, st_ref, g_ref, b_ref, o_ref,
                         *, shifts, m, count):
    xt = jnp.transpose(x_ref[0], (1, 0))
    acc = _conv_acc(xt, w_ref, shifts, m)
    tot = jnp.sum(st_ref[...], axis=0)
    mean = tot[0] / count
    var = jnp.maximum(tot[1] / count - mean * mean, 0.0)
    scale = g_ref[0] * jax.lax.rsqrt(var + _EPS)
    shift = b_ref[0] - mean * scale
    res = jnp.maximum(acc * scale[None, :] + shift[None, :], 0.0)
    o_ref[0] = jnp.transpose(res, (1, 0))


def kernel(x_nchw, conv_w, conv_b, gamma, beta):
    del conv_b
    N, Cin, H, W = x_nchw.shape
    Cout, cin2, kh, kw = conv_w.shape
    assert cin2 == Cin
    ph = pw = 1
    Hp, Wp = H + 2 * ph, W + 2 * pw
    Ho, Wo = Hp - kh + 1, Wp - kw + 1

    Mimg = Hp * Wp
    m = Ho * Wp
    m = (m + 7) // 8 * 8
    halo = (kh - 1) * Wp + (kw - 1)
    rows = (m + halo + 127) // 128 * 128
    shifts = [di * Wp + dj for di in range(kh) for dj in range(kw)]
    count = float(N * Ho * Wo)

    xp = jnp.pad(x_nchw.astype(jnp.bfloat16),
                 ((0, 0), (0, 0), (ph, ph), (pw, pw)))
    xf = xp.reshape(N, Cin, Mimg)
    xf = jnp.pad(xf, ((0, 0), (0, 0), (0, rows - Mimg)))

    w9 = jnp.transpose(conv_w, (2, 3, 1, 0)).reshape(kh * kw, Cin, Cout)
    w9 = w9.astype(jnp.bfloat16)

    r = jnp.arange(m)
    valid = (r < Ho * Wp) & ((r % Wp) < Wo)
    mask = valid.astype(jnp.float32).reshape(m, 1)

    cparams = pltpu.CompilerParams(
        dimension_semantics=("parallel",),
        vmem_limit_bytes=100 * 1024 * 1024,
    )

    stats = pl.pallas_call(
        functools.partial(_stats_kernel, shifts=shifts, m=m),
        grid=(N,),
        in_specs=[
            pl.BlockSpec((1, Cin, rows), lambda n: (n, 0, 0)),
            pl.BlockSpec((kh * kw, Cin, Cout), lambda n: (0, 0, 0)),
            pl.BlockSpec((m, 1), lambda n: (0, 0)),
        ],
        out_specs=pl.BlockSpec((1, 2, Cout), lambda n: (n, 0, 0)),
        out_shape=jax.ShapeDtypeStruct((N, 2, Cout), jnp.float32),
        compiler_params=cparams,
    )(xf, w9, mask)

    out_flat = pl.pallas_call(
        functools.partial(_conv_bn_relu_kernel, shifts=shifts, m=m,
                          count=count),
        grid=(N,),
        in_specs=[
            pl.BlockSpec((1, Cin, rows), lambda n: (n, 0, 0)),
            pl.BlockSpec((kh * kw, Cin, Cout), lambda n: (0, 0, 0)),
            pl.BlockSpec((N, 2, Cout), lambda n: (0, 0, 0)),
            pl.BlockSpec((1, Cout), lambda n: (0, 0)),
            pl.BlockSpec((1, Cout), lambda n: (0, 0)),
        ],
        out_specs=pl.BlockSpec((1, Cout, m), lambda n: (n, 0, 0)),
        out_shape=jax.ShapeDtypeStruct((N, Cout, m), jnp.float32),
        compiler_params=cparams,
    )(xf, w9, stats, gamma.astype(jnp.float32).reshape(1, Cout),
      beta.astype(jnp.float32).reshape(1, Cout))

    out = out_flat[:, :, :Ho * Wp].reshape(N, Cout, Ho, Wp)[:, :, :, :Wo]
    return out


# speedup vs baseline: 2.0228x; 2.0228x over previous
import functools

import jax
import jax.numpy as jnp
from jax.experimental import pallas as pl
from jax.experimental.pallas import tpu as pltpu

_EPS = 1e-5


def _conv_columns(x_ref, w_ref, *, kh, kw, wo, hw, pad_rows):
    cin = x_ref.shape[1]
    xt = jnp.transpose(x_ref[0], (1, 0))
    zpad = jnp.zeros((pad_rows, cin), dtype=xt.dtype)
    xe = jnp.concatenate([zpad, xt, zpad], axis=0)

    parts = []
    for dj in range(kw):
        acc = None
        for di in range(kh):
            s = pad_rows + (di - (kh // 2)) * wo + (dj - (kw // 2))
            p = jnp.dot(xe[s:s + hw, :], w_ref[di * kw + dj],
                        preferred_element_type=jnp.float32)
            acc = p if acc is None else acc + p
        parts.append(acc)
    return parts


def _edge_masked_sum(parts, *, kw, wo, hw):
    col = jax.lax.broadcasted_iota(jnp.int32, (hw, 1), 0) % wo
    acc = parts[kw // 2]
    for dj in range(kw):
        if dj == kw // 2:
            continue
        off = dj - (kw // 2)
        if off < 0:
            good = (col >= -off).astype(jnp.float32)
        else:
            good = (col < wo - off).astype(jnp.float32)
        acc = acc + parts[dj] * good
    return acc


def _stats_kernel(x_ref, w_ref, s_ref, *, kh, kw, wo, hw, pad_rows):
    parts = _conv_columns(x_ref, w_ref, kh=kh, kw=kw, wo=wo, hw=hw,
                          pad_rows=pad_rows)
    acc = _edge_masked_sum(parts, kw=kw, wo=wo, hw=hw)
    s1 = jnp.sum(acc, axis=0, keepdims=True)
    s2 = jnp.sum(acc * acc, axis=0, keepdims=True)
    s_ref[0] = jnp.concatenate([s1, s2], axis=0)


def _conv_bn_relu_kernel(x_ref, w_ref, st_ref, g_ref, b_ref, o_ref,
                         *, kh, kw, wo, hw, pad_rows, count):
    parts = _conv_columns(x_ref, w_ref, kh=kh, kw=kw, wo=wo, hw=hw,
                          pad_rows=pad_rows)
    acc = _edge_masked_sum(parts, kw=kw, wo=wo, hw=hw)
    tot = jnp.sum(st_ref[...], axis=0)
    mean = tot[0] / count
    var = jnp.maximum(tot[1] / count - mean * mean, 0.0)
    scale = g_ref[0] * jax.lax.rsqrt(var + _EPS)
    shift = b_ref[0] - mean * scale
    res = jnp.maximum(acc * scale[None, :] + shift[None, :], 0.0)
    o_ref[0] = jnp.transpose(res, (1, 0))


def kernel(x_nchw, conv_w, conv_b, gamma, beta):
    del conv_b
    N, Cin, H, W = x_nchw.shape
    Cout, cin2, kh, kw = conv_w.shape
    assert cin2 == Cin
    Ho, Wo = H, W
    hw = H * W
    pad_rows = (W + kw // 2 + 7) // 8 * 8
    count = float(N * Ho * Wo)

    xf = x_nchw.astype(jnp.float32).reshape(N, Cin, hw)
    w9 = jnp.transpose(conv_w, (2, 3, 1, 0)).reshape(kh * kw, Cin, Cout)
    w9 = w9.astype(jnp.float32)

    cparams = pltpu.CompilerParams(
        dimension_semantics=("parallel",),
        vmem_limit_bytes=100 * 1024 * 1024,
    )

    stats = pl.pallas_call(
        functools.partial(_stats_kernel, kh=kh, kw=kw, wo=Wo, hw=hw,
                          pad_rows=pad_rows),
        grid=(N,),
        in_specs=[
            pl.BlockSpec((1, Cin, hw), lambda n: (n, 0, 0)),
            pl.BlockSpec((kh * kw, Cin, Cout), lambda n: (0, 0, 0)),
        ],
        out_specs=pl.BlockSpec((1, 2, Cout), lambda n: (n, 0, 0)),
        out_shape=jax.ShapeDtypeStruct((N, 2, Cout), jnp.float32),
        compiler_params=cparams,
    )(xf, w9)

    out_flat = pl.pallas_call(
        functools.partial(_conv_bn_relu_kernel, kh=kh, kw=kw, wo=Wo, hw=hw,
                          pad_rows=pad_rows, count=count),
        grid=(N,),
        in_specs=[
            pl.BlockSpec((1, Cin, hw), lambda n: (n, 0, 0)),
            pl.BlockSpec((kh * kw, Cin, Cout), lambda n: (0, 0, 0)),
            pl.BlockSpec((N, 2, Cout), lambda n: (0, 0, 0)),
            pl.BlockSpec((1, Cout), lambda n: (0, 0)),
            pl.BlockSpec((1, Cout), lambda n: (0, 0)),
        ],
        out_specs=pl.BlockSpec((1, Cout, hw), lambda n: (n, 0, 0)),
        out_shape=jax.ShapeDtypeStruct((N, Cout, hw), jnp.float32),
        compiler_params=cparams,
    )(xf, w9, stats, gamma.astype(jnp.float32).reshape(1, Cout),
      beta.astype(jnp.float32).reshape(1, Cout))

    return out_flat.reshape(N, Cout, Ho, Wo)
